# baseline (speedup 1.0000x reference)
import functools

import jax
import jax.numpy as jnp
from jax import lax
from jax.experimental import pallas as pl
from jax.experimental.pallas import tpu as pltpu
from jax.experimental.pallas import tpu_sc as plsc

NC = 2
NS = 16
L = 16
NW = NC * NS
BATCH = 16384
BPW = BATCH // NW
NCHUNK = 4
CHUNK = BPW // NCHUNK
DIM = 64


def _body(data_hbm, user_hbm, item_hbm, out_hbm,
          d_v, uidx_v, vidx_v, urows_v, vrows_v, out_v, sem):
    cid = lax.axis_index("c")
    sid = lax.axis_index("s")
    wid = sid * NC + cid
    base = wid * BPW

    pltpu.sync_copy(data_hbm.at[pl.ds(base * 2, BPW * 2)], d_v)

    iota = lax.iota(jnp.int32, L)

    for j in range(NCHUNK):
        urow = uidx_v.at[j]
        irow = vidx_v.at[j]
        for k in range(CHUNK // L):
            flat = (iota + (j * CHUNK + k * L)) * 2
            u16 = plsc.load_gather(d_v, [flat])
            i16 = plsc.load_gather(d_v, [flat + 1])
            urow[pl.ds(k * L, L)] = u16
            irow[pl.ds(k * L, L)] = i16

    copies = []
    for j in range(NCHUNK):
        copies.append(
            pltpu.async_copy(user_hbm.at[uidx_v.at[j]], urows_v.at[j], sem))
        copies.append(
            pltpu.async_copy(item_hbm.at[vidx_v.at[j]], vrows_v.at[j], sem))
    for cp in copies:
        cp.wait()

    for j in range(NCHUNK):
        jsplat = jnp.full((L,), j, jnp.int32)

        def grp(g, _, j=j, jsplat=jsplat):
            rows = iota + g * L
            accs = [jnp.zeros((L,), jnp.float32) for _ in range(4)]
            for d in range(DIM):
                dsplat = jnp.full((L,), d, jnp.int32)
                uu = plsc.load_gather(urows_v, [jsplat, rows, dsplat])
                vv = plsc.load_gather(vrows_v, [jsplat, rows, dsplat])
                accs[d % 4] = accs[d % 4] + uu * vv
            acc = (accs[0] + accs[1]) + (accs[2] + accs[3])
            res = 1.0 / (1.0 + jnp.exp(-acc))
            out_v[pl.ds(j * CHUNK + g * L, L)] = res
            return 0

        lax.fori_loop(0, CHUNK // L, grp, 0)

    pltpu.sync_copy(out_v, out_hbm.at[pl.ds(base, BPW)])


@jax.jit
def kernel(data, user_embeds, item_embeds):
    mesh = plsc.VectorSubcoreMesh(core_axis_name="c", subcore_axis_name="s")
    f = functools.partial(
        pl.kernel,
        out_type=jax.ShapeDtypeStruct((BATCH,), jnp.float32),
        mesh=mesh,
        scratch_types=[
            pltpu.VMEM((BPW * 2,), jnp.int32),
            pltpu.VMEM((NCHUNK, CHUNK), jnp.int32),
            pltpu.VMEM((NCHUNK, CHUNK), jnp.int32),
            pltpu.VMEM((NCHUNK, CHUNK, DIM), jnp.float32),
            pltpu.VMEM((NCHUNK, CHUNK, DIM), jnp.float32),
            pltpu.VMEM((BPW,), jnp.float32),
            pltpu.SemaphoreType.DMA,
        ],
        compiler_params=pltpu.CompilerParams(
            needs_layout_passes=False, use_tc_tiling_on_sc=False),
    )(_body)
    return f(data.reshape(-1), user_embeds, item_embeds)


# speedup vs baseline: 1.1582x; 1.1582x over previous
import functools

import jax
import jax.numpy as jnp
from jax import lax
from jax.experimental import pallas as pl
from jax.experimental.pallas import tpu as pltpu
from jax.experimental.pallas import tpu_sc as plsc

NC = 2
NS = 16
L = 16
NW = NC * NS
BATCH = 16384
BPW = BATCH // NW
NCHUNK = 4
CHUNK = BPW // NCHUNK
DIM = 64


def _body(data_hbm, user_hbm, item_hbm, out_hbm,
          d_v, uidx_v, vidx_v, urows_v, vrows_v, out_v, sem):
    cid = lax.axis_index("c")
    sid = lax.axis_index("s")
    wid = sid * NC + cid
    base = wid * BPW

    pltpu.sync_copy(data_hbm.at[pl.ds(base * 2, BPW * 2)], d_v)

    iota = lax.iota(jnp.int32, L)

    for j in range(NCHUNK):
        urow = uidx_v.at[j]
        irow = vidx_v.at[j]
        for k in range(CHUNK // L):
            flat = (iota + (j * CHUNK + k * L)) * 2
            u16 = plsc.load_gather(d_v, [flat])
            i16 = plsc.load_gather(d_v, [flat + 1])
            urow[pl.ds(k * L, L)] = u16
            irow[pl.ds(k * L, L)] = i16

    copies = []
    for j in range(NCHUNK):
        copies.append(
            pltpu.async_copy(user_hbm.at[uidx_v.at[j]], urows_v.at[j], sem))
        copies.append(
            pltpu.async_copy(item_hbm.at[vidx_v.at[j]], vrows_v.at[j], sem))
    for cp in copies:
        cp.wait()

    for j in range(NCHUNK):
        jsplat = jnp.full((L,), j, jnp.int32)

        def grp(g, _, j=j, jsplat=jsplat):
            rows = iota + g * L
            accs = [jnp.zeros((L,), jnp.float32) for _ in range(4)]
            for d in range(DIM):
                didx = (iota + d) & (DIM - 1)
                uu = plsc.load_gather(urows_v, [jsplat, rows, didx])
                vv = plsc.load_gather(vrows_v, [jsplat, rows, didx])
                accs[d % 4] = accs[d % 4] + uu * vv
            acc = (accs[0] + accs[1]) + (accs[2] + accs[3])
            res = 1.0 / (1.0 + jnp.exp(-acc))
            out_v[pl.ds(j * CHUNK + g * L, L)] = res
            return 0

        lax.fori_loop(0, CHUNK // L, grp, 0)

    pltpu.sync_copy(out_v, out_hbm.at[pl.ds(base, BPW)])


@jax.jit
def kernel(data, user_embeds, item_embeds):
    mesh = plsc.VectorSubcoreMesh(core_axis_name="c", subcore_axis_name="s")
    f = functools.partial(
        pl.kernel,
        out_type=jax.ShapeDtypeStruct((BATCH,), jnp.float32),
        mesh=mesh,
        scratch_types=[
            pltpu.VMEM((BPW * 2,), jnp.int32),
            pltpu.VMEM((NCHUNK, CHUNK), jnp.int32),
            pltpu.VMEM((NCHUNK, CHUNK), jnp.int32),
            pltpu.VMEM((NCHUNK, CHUNK, DIM), jnp.float32),
            pltpu.VMEM((NCHUNK, CHUNK, DIM), jnp.float32),
            pltpu.VMEM((BPW,), jnp.float32),
            pltpu.SemaphoreType.DMA,
        ],
        compiler_params=pltpu.CompilerParams(
            needs_layout_passes=False, use_tc_tiling_on_sc=False),
    )(_body)
    return f(data.reshape(-1), user_embeds, item_embeds)


# speedup vs baseline: 1.1814x; 1.0200x over previous
import functools

import jax
import jax.numpy as jnp
from jax import lax
from jax.experimental import pallas as pl
from jax.experimental.pallas import tpu as pltpu
from jax.experimental.pallas import tpu_sc as plsc

NC = 2
NS = 16
L = 16
NW = NC * NS
BATCH = 16384
BPW = BATCH // NW
NCHUNK = 4
CHUNK = BPW // NCHUNK
DIM = 64
ROW = 2 * DIM


def _body(data_hbm, user_hbm, item_hbm, out_hbm,
          d_v, ugidx_v, vgidx_v, uhalf_v, vhalf_v,
          urows_v, vrows_v, out_v, sem0, sem1):
    cid = lax.axis_index("c")
    sid = lax.axis_index("s")
    wid = sid * NC + cid
    base = wid * BPW

    pltpu.sync_copy(data_hbm.at[pl.ds(base * 2, BPW * 2)], d_v)

    iota = lax.iota(jnp.int32, L)

    for j in range(NCHUNK):
        urow = ugidx_v.at[j]
        irow = vgidx_v.at[j]
        for k in range(CHUNK // L):
            off = j * CHUNK + k * L
            flat = (iota + off) * 2
            u16 = plsc.load_gather(d_v, [flat])
            i16 = plsc.load_gather(d_v, [flat + 1])
            urow[pl.ds(k * L, L)] = u16 >> 1
            irow[pl.ds(k * L, L)] = i16 >> 1
            uhalf_v[pl.ds(off, L)] = (u16 & 1) << 6
            vhalf_v[pl.ds(off, L)] = (i16 & 1) << 6

    sems = (sem0, sem1)

    def fire(j):
        b = j % 2
        return (
            pltpu.async_copy(user_hbm.at[ugidx_v.at[j]], urows_v.at[b],
                             sems[b]),
            pltpu.async_copy(item_hbm.at[vgidx_v.at[j]], vrows_v.at[b],
                             sems[b]),
        )

    copies = [None] * NCHUNK
    copies[0] = fire(0)
    for j in range(NCHUNK):
        if j + 1 < NCHUNK:
            copies[j + 1] = fire(j + 1)
        copies[j][0].wait()
        copies[j][1].wait()

        jb = j % 2
        jsplat = jnp.full((L,), jb, jnp.int32)

        def grp(g, _, j=j, jsplat=jsplat):
            rows = iota + g * L
            uh = uhalf_v[pl.ds(j * CHUNK + g * L, L)]
            vh = vhalf_v[pl.ds(j * CHUNK + g * L, L)]
            def dblk(d0, accs):
                accs = list(accs)
                for t in range(DIM // 2):
                    didx = (iota + (d0 * (DIM // 2) + t)) & (DIM - 1)
                    uu = plsc.load_gather(urows_v, [jsplat, rows, uh + didx])
                    vv = plsc.load_gather(vrows_v, [jsplat, rows, vh + didx])
                    accs[t % 4] = accs[t % 4] + uu * vv
                return tuple(accs)

            accs = lax.fori_loop(
                0, 2, dblk,
                tuple(jnp.zeros((L,), jnp.float32) for _ in range(4)))
            acc = (accs[0] + accs[1]) + (accs[2] + accs[3])
            res = 1.0 / (1.0 + jnp.exp(-acc))
            out_v[pl.ds(j * CHUNK + g * L, L)] = res
            return 0

        lax.fori_loop(0, CHUNK // L, grp, 0)

    pltpu.sync_copy(out_v, out_hbm.at[pl.ds(base, BPW)])


@jax.jit
def kernel(data, user_embeds, item_embeds):
    mesh = plsc.VectorSubcoreMesh(core_axis_name="c", subcore_axis_name="s")
    f = functools.partial(
        pl.kernel,
        out_type=jax.ShapeDtypeStruct((BATCH,), jnp.float32),
        mesh=mesh,
        scratch_types=[
            pltpu.VMEM((BPW * 2,), jnp.int32),
            pltpu.VMEM((NCHUNK, CHUNK), jnp.int32),
            pltpu.VMEM((NCHUNK, CHUNK), jnp.int32),
            pltpu.VMEM((BPW,), jnp.int32),
            pltpu.VMEM((BPW,), jnp.int32),
            pltpu.VMEM((2, CHUNK, ROW), jnp.float32),
            pltpu.VMEM((2, CHUNK, ROW), jnp.float32),
            pltpu.VMEM((BPW,), jnp.float32),
            pltpu.SemaphoreType.DMA,
            pltpu.SemaphoreType.DMA,
        ],
        compiler_params=pltpu.CompilerParams(
            needs_layout_passes=False, use_tc_tiling_on_sc=True),
    )(_body)
    return f(data.reshape(-1),
             user_embeds.reshape(DIM * 100000 // ROW, ROW),
             item_embeds.reshape(DIM * 100000 // ROW, ROW))
